# Initial kernel scaffold; baseline (speedup 1.0000x reference)
import jax
import jax.numpy as jnp
from jax.experimental import pallas as pl


def kernel(x, pos_table):
    raise NotImplementedError("write your pallas kernel here")


# speedup vs baseline: 1.2284x; 1.2284x over previous
import jax
import jax.numpy as jnp
from jax.experimental import pallas as pl

BS = 512


def _body(x_ref, pos_ref, out_ref):
    out_ref[...] = x_ref[...] + pos_ref[...][None]


def kernel(x, pos_table):
    b, s, d = x.shape
    pos = pos_table[:s]
    grid = (s // BS, b)
    return pl.pallas_call(
        _body,
        grid=grid,
        in_specs=[
            pl.BlockSpec((1, BS, d), lambda i, j: (j, i, 0)),
            pl.BlockSpec((BS, d), lambda i, j: (i, 0)),
        ],
        out_specs=pl.BlockSpec((1, BS, d), lambda i, j: (j, i, 0)),
        out_shape=jax.ShapeDtypeStruct((b, s, d), x.dtype),
    )(x, pos)
